# baseline (speedup 1.0000x reference)
import functools

import jax
import jax.numpy as jnp
from jax import lax
from jax.experimental import pallas as pl
from jax.experimental.pallas import tpu as pltpu
from jax.experimental.pallas import tpu_sc as plsc

N = 10000
D = 128
E = 320000
HALF = 64
NS = 16
CHUNK = 128
CPT = 160
NCHUNK = CPT * NS
E_PAD = NCHUNK * CHUNK
NBLK = 4
CPB = CPT // NBLK
ROWS_PT = N // NS
NBUF = 8


def _agg_body(h_hbm, src_hbm, dst_hbm, out_hbm,
              agg_sh, sidx_v, didx_v,
              buf0, buf1, buf2, buf3, buf4, buf5, buf6, buf7,
              gsems, ssems):
    c = lax.axis_index("c")
    s = lax.axis_index("s")
    r0 = s * ROWS_PT
    bufs = (buf0, buf1, buf2, buf3, buf4, buf5, buf6, buf7)

    pltpu.sync_copy(h_hbm.at[c, pl.ds(r0, ROWS_PT)],
                    agg_sh.at[pl.ds(r0, ROWS_PT)])
    plsc.subcore_barrier()

    def gather(j, t):
        return pltpu.async_copy(h_hbm.at[c].at[sidx_v.at[j]], bufs[t],
                                gsems.at[t])

    def gather_wait(j, t):
        pltpu.make_async_copy(h_hbm.at[c].at[sidx_v.at[j]], bufs[t],
                              gsems.at[t]).wait()

    def scatter(j, t):
        return pltpu.async_copy(bufs[t], agg_sh.at[didx_v.at[j]],
                                ssems.at[t], add=True)

    def scatter_wait(j, t):
        pltpu.make_async_copy(bufs[t], agg_sh.at[didx_v.at[j]],
                              ssems.at[t]).wait()

    def blk(bi, carry):
        ch0 = s * CPT + bi * CPB
        pltpu.sync_copy(src_hbm.at[pl.ds(ch0, CPB)], sidx_v)
        pltpu.sync_copy(dst_hbm.at[pl.ds(ch0, CPB)], didx_v)

        def body(q, carry2):
            j = NBUF * q
            for t in range(NBUF):
                gather(j + t, t)
            for t in range(NBUF):
                gather_wait(j + t, t)
                scatter(j + t, t)
            for t in range(NBUF):
                scatter_wait(j + t, t)
            return carry2

        lax.fori_loop(0, CPB // NBUF, body, 0)
        return carry

    lax.fori_loop(0, NBLK, blk, 0)
    plsc.subcore_barrier()

    pltpu.sync_copy(agg_sh.at[pl.ds(r0, ROWS_PT)],
                    out_hbm.at[c, pl.ds(r0, ROWS_PT)])


_agg = pl.kernel(
    _agg_body,
    out_type=jax.ShapeDtypeStruct((2, N, HALF), jnp.float32),
    mesh=plsc.VectorSubcoreMesh(core_axis_name="c", subcore_axis_name="s"),
    scratch_types=[
        pltpu.VMEM_SHARED((N + 8, HALF), jnp.float32),
        pltpu.VMEM((CPB, CHUNK), jnp.int32),
        pltpu.VMEM((CPB, CHUNK), jnp.int32),
        pltpu.VMEM((CHUNK, HALF), jnp.float32),
        pltpu.VMEM((CHUNK, HALF), jnp.float32),
        pltpu.VMEM((CHUNK, HALF), jnp.float32),
        pltpu.VMEM((CHUNK, HALF), jnp.float32),
        pltpu.VMEM((CHUNK, HALF), jnp.float32),
        pltpu.VMEM((CHUNK, HALF), jnp.float32),
        pltpu.VMEM((CHUNK, HALF), jnp.float32),
        pltpu.VMEM((CHUNK, HALF), jnp.float32),
        pltpu.SemaphoreType.DMA((NBUF,)),
        pltpu.SemaphoreType.DMA((NBUF,)),
    ],
    compiler_params=pltpu.CompilerParams(use_tc_tiling_on_sc=False),
)


def _mlp_body(z_ref, w_ref, b_ref, o_ref, *, relu, split_out):
    z = jnp.concatenate([z_ref[0], z_ref[1]], axis=-1)
    acc = jnp.dot(z, w_ref[...], preferred_element_type=jnp.float32)
    acc = acc + b_ref[...]
    if relu:
        acc = jnp.maximum(acc, 0.0)
    if split_out:
        o_ref[0] = acc[:, :HALF]
        o_ref[1] = acc[:, HALF:]
    else:
        o_ref[...] = acc


def _mlp(z2, w, b, relu, split_out):
    blk = 1000
    if split_out:
        out_shape = jax.ShapeDtypeStruct((2, N, HALF), jnp.float32)
        out_spec = pl.BlockSpec((2, blk, HALF), lambda i: (0, i, 0))
    else:
        out_shape = jax.ShapeDtypeStruct((N, D), jnp.float32)
        out_spec = pl.BlockSpec((blk, D), lambda i: (i, 0))
    return pl.pallas_call(
        functools.partial(_mlp_body, relu=relu, split_out=split_out),
        grid=(N // blk,),
        in_specs=[
            pl.BlockSpec((2, blk, HALF), lambda i: (0, i, 0)),
            pl.BlockSpec((D, D), lambda i: (0, 0)),
            pl.BlockSpec((1, D), lambda i: (0, 0)),
        ],
        out_specs=out_spec,
        out_shape=out_shape,
    )(z2, w, b.reshape(1, D))


def kernel(x, edge_index, W1, b1, W2, b2, W3, b3):
    ei = edge_index.astype(jnp.int32)
    pad = E_PAD - E
    src = jnp.concatenate([ei[0], jnp.zeros((pad,), jnp.int32)])
    dst = jnp.concatenate([ei[1], jnp.full((pad,), N, jnp.int32)])
    src = src.reshape(NCHUNK, CHUNK)
    dst = dst.reshape(NCHUNK, CHUNK)

    h2 = jnp.stack([x[:, :HALF], x[:, HALF:]])
    z2 = _agg(h2, src, dst)
    h2 = _mlp(z2, W1, b1, True, True)
    z2 = _agg(h2, src, dst)
    h2 = _mlp(z2, W2, b2, True, True)
    z2 = _agg(h2, src, dst)
    return _mlp(z2, W3, b3, False, False)


# speedup vs baseline: 1.6522x; 1.6522x over previous
import functools

import jax
import jax.numpy as jnp
from jax import lax
from jax.experimental import pallas as pl
from jax.experimental.pallas import tpu as pltpu
from jax.experimental.pallas import tpu_sc as plsc

N = 10000
D = 128
E = 320000
HALF = 64
NS = 16
CHUNK = 128
CPT = 160
NCHUNK = CPT * NS
E_PAD = NCHUNK * CHUNK
NBLK = 4
CPB = CPT // NBLK
ROWS_PT = N // NS
NBUF = 4


def _agg_body(h_hbm, h2_hbm, src_hbm, dst_hbm, out_hbm,
              h_sh, agg_sh, sidx_v, didx_v,
              buf0, buf1, buf2, buf3, gsems, ssems):
    c = lax.axis_index("c")
    s = lax.axis_index("s")
    c0 = c * HALF
    r0 = s * ROWS_PT
    bufs = (buf0, buf1, buf2, buf3)

    pltpu.sync_copy(h_hbm.at[pl.ds(r0, ROWS_PT), pl.ds(c0, HALF)],
                    h_sh.at[pl.ds(r0, ROWS_PT)])
    pltpu.sync_copy(h_hbm.at[pl.ds(r0, ROWS_PT), pl.ds(c0, HALF)],
                    agg_sh.at[pl.ds(r0, ROWS_PT)])
    plsc.subcore_barrier()

    def gather_sp(j, t):
        return pltpu.async_copy(h_sh.at[sidx_v.at[j]], bufs[t], gsems.at[t])

    def gather_hbm(j, t):
        return pltpu.async_copy(h2_hbm.at[c].at[sidx_v.at[j]], bufs[t],
                                gsems.at[t])

    def gather_wait(j, t):
        pltpu.make_async_copy(h_sh.at[sidx_v.at[j]], bufs[t],
                              gsems.at[t]).wait()

    def scatter(j, t):
        return pltpu.async_copy(bufs[t], agg_sh.at[didx_v.at[j]],
                                ssems.at[t], add=True)

    def scatter_wait(j, t):
        pltpu.make_async_copy(bufs[t], agg_sh.at[didx_v.at[j]],
                              ssems.at[t]).wait()

    def group(j, srcmap):
        for t in range(NBUF):
            if srcmap[t] == "H":
                gather_hbm(j + t, t)
        for t in range(NBUF):
            if srcmap[t] == "S":
                gather_sp(j + t, t)
        for t in range(NBUF):
            gather_wait(j + t, t)
            scatter(j + t, t)
        for t in range(NBUF):
            scatter_wait(j + t, t)

    def blk(bi, carry):
        ch0 = s * CPT + bi * CPB
        pltpu.sync_copy(src_hbm.at[pl.ds(ch0, CPB)], sidx_v)
        pltpu.sync_copy(dst_hbm.at[pl.ds(ch0, CPB)], didx_v)

        def body(q, carry2):
            j = 2 * NBUF * q
            group(j, "SSSH")
            group(j + NBUF, "SSHH")
            return carry2

        lax.fori_loop(0, CPB // (2 * NBUF), body, 0)
        return carry

    lax.fori_loop(0, NBLK, blk, 0)
    plsc.subcore_barrier()

    pltpu.sync_copy(agg_sh.at[pl.ds(r0, ROWS_PT)],
                    out_hbm.at[pl.ds(r0, ROWS_PT), pl.ds(c0, HALF)])


_agg = pl.kernel(
    _agg_body,
    out_type=jax.ShapeDtypeStruct((N, D), jnp.float32),
    mesh=plsc.VectorSubcoreMesh(core_axis_name="c", subcore_axis_name="s"),
    scratch_types=[
        pltpu.VMEM_SHARED((N, HALF), jnp.float32),
        pltpu.VMEM_SHARED((N + 8, HALF), jnp.float32),
        pltpu.VMEM((CPB, CHUNK), jnp.int32),
        pltpu.VMEM((CPB, CHUNK), jnp.int32),
        pltpu.VMEM((CHUNK, HALF), jnp.float32),
        pltpu.VMEM((CHUNK, HALF), jnp.float32),
        pltpu.VMEM((CHUNK, HALF), jnp.float32),
        pltpu.VMEM((CHUNK, HALF), jnp.float32),
        pltpu.SemaphoreType.DMA((NBUF,)),
        pltpu.SemaphoreType.DMA((NBUF,)),
    ],
    compiler_params=pltpu.CompilerParams(use_tc_tiling_on_sc=False),
)


def _mlp_body(z_ref, w_ref, b_ref, o_ref, o2_ref=None, *, relu):
    acc = jnp.dot(z_ref[...], w_ref[...],
                  preferred_element_type=jnp.float32) + b_ref[...]
    if relu:
        acc = jnp.maximum(acc, 0.0)
    o_ref[...] = acc
    if o2_ref is not None:
        o2_ref[0] = acc[:, :HALF]
        o2_ref[1] = acc[:, HALF:]


def _mlp(z, w, b, relu, split_out):
    blk = 1000
    out_shape = [jax.ShapeDtypeStruct((N, D), jnp.float32)]
    out_specs = [pl.BlockSpec((blk, D), lambda i: (i, 0))]
    if split_out:
        out_shape.append(jax.ShapeDtypeStruct((2, N, HALF), jnp.float32))
        out_specs.append(pl.BlockSpec((2, blk, HALF), lambda i: (0, i, 0)))
    return pl.pallas_call(
        functools.partial(_mlp_body, relu=relu),
        grid=(N // blk,),
        in_specs=[
            pl.BlockSpec((blk, D), lambda i: (i, 0)),
            pl.BlockSpec((D, D), lambda i: (0, 0)),
            pl.BlockSpec((1, D), lambda i: (0, 0)),
        ],
        out_specs=out_specs,
        out_shape=out_shape,
    )(z, w, b.reshape(1, D))


def kernel(x, edge_index, W1, b1, W2, b2, W3, b3):
    ei = edge_index.astype(jnp.int32)
    pad = E_PAD - E
    src = jnp.concatenate([ei[0], jnp.zeros((pad,), jnp.int32)])
    dst = jnp.concatenate([ei[1], jnp.full((pad,), N, jnp.int32)])
    src = src.reshape(NCHUNK, CHUNK)
    dst = dst.reshape(NCHUNK, CHUNK)

    h = x
    h2 = jnp.stack([x[:, :HALF], x[:, HALF:]])
    z = _agg(h, h2, src, dst)
    h, h2 = _mlp(z, W1, b1, True, True)
    z = _agg(h, h2, src, dst)
    h, h2 = _mlp(z, W2, b2, True, True)
    z = _agg(h, h2, src, dst)
    (out,) = _mlp(z, W3, b3, False, False)
    return out


# speedup vs baseline: 1.7093x; 1.0346x over previous
import functools

import jax
import jax.numpy as jnp
from jax import lax
from jax.experimental import pallas as pl
from jax.experimental.pallas import tpu as pltpu
from jax.experimental.pallas import tpu_sc as plsc

N = 10000
D = 128
E = 320000
HALF = 64
NS = 16
CHUNK = 128
CPT = 160
NCHUNK = CPT * NS
E_PAD = NCHUNK * CHUNK
NBLK = 8
CPB = CPT // NBLK
ROWS_PT = N // NS
NBUF = 5


def _agg_body(h_hbm, src_hbm, dst_hbm, out_hbm,
              h_sh, agg_sh, sidx_v, didx_v,
              buf0, buf1, buf2, buf3, buf4, gsems, ssems):
    c = lax.axis_index("c")
    s = lax.axis_index("s")
    c0 = c * HALF
    r0 = s * ROWS_PT
    bufs = (buf0, buf1, buf2, buf3, buf4)

    pltpu.sync_copy(h_hbm.at[pl.ds(r0, ROWS_PT), pl.ds(c0, HALF)],
                    h_sh.at[pl.ds(r0, ROWS_PT)])
    pltpu.sync_copy(h_hbm.at[pl.ds(r0, ROWS_PT), pl.ds(c0, HALF)],
                    agg_sh.at[pl.ds(r0, ROWS_PT)])
    plsc.subcore_barrier()

    def gather(j, t):
        return pltpu.async_copy(h_sh.at[sidx_v.at[j]], bufs[t], gsems.at[t])

    def gather_wait(j, t):
        pltpu.make_async_copy(h_sh.at[sidx_v.at[j]], bufs[t],
                              gsems.at[t]).wait()

    def scatter(j, t):
        return pltpu.async_copy(bufs[t], agg_sh.at[didx_v.at[j]],
                                ssems.at[t], add=True)

    def scatter_wait(j, t):
        pltpu.make_async_copy(bufs[t], agg_sh.at[didx_v.at[j]],
                              ssems.at[t]).wait()

    def blk(bi, carry):
        ch0 = s * CPT + bi * CPB
        pltpu.sync_copy(src_hbm.at[pl.ds(ch0, CPB)], sidx_v)
        pltpu.sync_copy(dst_hbm.at[pl.ds(ch0, CPB)], didx_v)

        def body(q, carry2):
            j = NBUF * q
            for t in range(NBUF):
                gather(j + t, t)
            for t in range(NBUF):
                gather_wait(j + t, t)
                scatter(j + t, t)
            for t in range(NBUF):
                scatter_wait(j + t, t)
            return carry2

        lax.fori_loop(0, CPB // NBUF, body, 0)
        return carry

    lax.fori_loop(0, NBLK, blk, 0)
    plsc.subcore_barrier()

    pltpu.sync_copy(agg_sh.at[pl.ds(r0, ROWS_PT)],
                    out_hbm.at[pl.ds(r0, ROWS_PT), pl.ds(c0, HALF)])


_agg = pl.kernel(
    _agg_body,
    out_type=jax.ShapeDtypeStruct((N, D), jnp.float32),
    mesh=plsc.VectorSubcoreMesh(core_axis_name="c", subcore_axis_name="s"),
    scratch_types=[
        pltpu.VMEM_SHARED((N, HALF), jnp.float32),
        pltpu.VMEM_SHARED((N + 8, HALF), jnp.float32),
        pltpu.VMEM((CPB, CHUNK), jnp.int32),
        pltpu.VMEM((CPB, CHUNK), jnp.int32),
        pltpu.VMEM((CHUNK, HALF), jnp.float32),
        pltpu.VMEM((CHUNK, HALF), jnp.float32),
        pltpu.VMEM((CHUNK, HALF), jnp.float32),
        pltpu.VMEM((CHUNK, HALF), jnp.float32),
        pltpu.VMEM((CHUNK, HALF), jnp.float32),
        pltpu.SemaphoreType.DMA((NBUF,)),
        pltpu.SemaphoreType.DMA((NBUF,)),
    ],
    compiler_params=pltpu.CompilerParams(use_tc_tiling_on_sc=False),
)


def _mlp_body(z_ref, w_ref, b_ref, o_ref, *, relu):
    acc = jnp.dot(z_ref[...], w_ref[...],
                  preferred_element_type=jnp.float32) + b_ref[...]
    o_ref[...] = jnp.maximum(acc, 0.0) if relu else acc


def _mlp(z, w, b, relu):
    blk = 1000
    return pl.pallas_call(
        functools.partial(_mlp_body, relu=relu),
        grid=(N // blk,),
        in_specs=[
            pl.BlockSpec((blk, D), lambda i: (i, 0)),
            pl.BlockSpec((D, D), lambda i: (0, 0)),
            pl.BlockSpec((1, D), lambda i: (0, 0)),
        ],
        out_specs=pl.BlockSpec((blk, D), lambda i: (i, 0)),
        out_shape=jax.ShapeDtypeStruct((N, D), jnp.float32),
    )(z, w, b.reshape(1, D))


def kernel(x, edge_index, W1, b1, W2, b2, W3, b3):
    ei = edge_index.astype(jnp.int32)
    pad = E_PAD - E
    src = jnp.concatenate([ei[0], jnp.zeros((pad,), jnp.int32)])
    dst = jnp.concatenate([ei[1], jnp.full((pad,), N, jnp.int32)])
    src = src.reshape(NCHUNK, CHUNK)
    dst = dst.reshape(NCHUNK, CHUNK)

    h = x
    z = _agg(h, src, dst)
    h = _mlp(z, W1, b1, True)
    z = _agg(h, src, dst)
    h = _mlp(z, W2, b2, True)
    z = _agg(h, src, dst)
    return _mlp(z, W3, b3, False)


# speedup vs baseline: 1.8441x; 1.0789x over previous
import functools

import jax
import jax.numpy as jnp
from jax import lax
from jax.experimental import pallas as pl
from jax.experimental.pallas import tpu as pltpu
from jax.experimental.pallas import tpu_sc as plsc

N = 10000
D = 128
E = 320000
HALF = 64
NS = 16
CHUNK = 128
CPT = 160
NCHUNK = CPT * NS
E_PAD = NCHUNK * CHUNK
NBLK = 4
CPB = CPT // NBLK
ROWS_PT = N // NS
NBUF = 4


def _agg_body(h_hbm, src_hbm, dst_hbm, out_hbm,
              h_sh, agg_sh, sidx_v, didx_v,
              buf0, buf1, buf2, buf3, gsems, ssems):
    c = lax.axis_index("c")
    s = lax.axis_index("s")
    r0 = s * ROWS_PT
    bufs = (buf0, buf1, buf2, buf3)

    pltpu.sync_copy(h_hbm.at[c, pl.ds(r0, ROWS_PT)],
                    h_sh.at[pl.ds(r0, ROWS_PT)])
    pltpu.sync_copy(h_hbm.at[c, pl.ds(r0, ROWS_PT)],
                    agg_sh.at[pl.ds(r0, ROWS_PT)])
    plsc.subcore_barrier()

    def gather(j, t):
        return pltpu.async_copy(h_sh.at[sidx_v.at[j]], bufs[t], gsems.at[t])

    def gather_wait(j, t):
        pltpu.make_async_copy(h_sh.at[sidx_v.at[j]], bufs[t],
                              gsems.at[t]).wait()

    def scatter(j, t):
        return pltpu.async_copy(bufs[t], agg_sh.at[didx_v.at[j]],
                                ssems.at[t], add=True)

    def scatter_wait(j, t):
        pltpu.make_async_copy(bufs[t], agg_sh.at[didx_v.at[j]],
                              ssems.at[t]).wait()

    def blk(bi, carry):
        ch0 = s * CPT + bi * CPB
        pltpu.sync_copy(src_hbm.at[pl.ds(ch0, CPB)], sidx_v)
        pltpu.sync_copy(dst_hbm.at[pl.ds(ch0, CPB)], didx_v)

        def body(q, carry2):
            j = NBUF * q
            for t in range(NBUF):
                gather(j + t, t)
            for t in range(NBUF):
                gather_wait(j + t, t)
                scatter(j + t, t)
            for t in range(NBUF):
                scatter_wait(j + t, t)
            return carry2

        lax.fori_loop(0, CPB // NBUF, body, 0)
        return carry

    lax.fori_loop(0, NBLK, blk, 0)
    plsc.subcore_barrier()

    pltpu.sync_copy(agg_sh.at[pl.ds(r0, ROWS_PT)],
                    out_hbm.at[c, pl.ds(r0, ROWS_PT)])


_agg = pl.kernel(
    _agg_body,
    out_type=jax.ShapeDtypeStruct((2, N, HALF), jnp.float32),
    mesh=plsc.VectorSubcoreMesh(core_axis_name="c", subcore_axis_name="s"),
    scratch_types=[
        pltpu.VMEM_SHARED((N, HALF), jnp.float32),
        pltpu.VMEM_SHARED((N + 8, HALF), jnp.float32),
        pltpu.VMEM((CPB, CHUNK), jnp.int32),
        pltpu.VMEM((CPB, CHUNK), jnp.int32),
        pltpu.VMEM((CHUNK, HALF), jnp.float32),
        pltpu.VMEM((CHUNK, HALF), jnp.float32),
        pltpu.VMEM((CHUNK, HALF), jnp.float32),
        pltpu.VMEM((CHUNK, HALF), jnp.float32),
        pltpu.SemaphoreType.DMA((NBUF,)),
        pltpu.SemaphoreType.DMA((NBUF,)),
    ],
    compiler_params=pltpu.CompilerParams(use_tc_tiling_on_sc=False),
)


def _mlp_body(z_ref, w_ref, b_ref, o_ref, *, relu, split_out):
    z = jnp.concatenate([z_ref[0], z_ref[1]], axis=-1)
    acc = jnp.dot(z, w_ref[...],
                  preferred_element_type=jnp.float32) + b_ref[...]
    if relu:
        acc = jnp.maximum(acc, 0.0)
    if split_out:
        o_ref[0] = acc[:, :HALF]
        o_ref[1] = acc[:, HALF:]
    else:
        o_ref[...] = acc


def _mlp(z2, w, b, relu, split_out):
    blk = 1000
    if split_out:
        out_shape = jax.ShapeDtypeStruct((2, N, HALF), jnp.float32)
        out_spec = pl.BlockSpec((2, blk, HALF), lambda i: (0, i, 0))
    else:
        out_shape = jax.ShapeDtypeStruct((N, D), jnp.float32)
        out_spec = pl.BlockSpec((blk, D), lambda i: (i, 0))
    return pl.pallas_call(
        functools.partial(_mlp_body, relu=relu, split_out=split_out),
        grid=(N // blk,),
        in_specs=[
            pl.BlockSpec((2, blk, HALF), lambda i: (0, i, 0)),
            pl.BlockSpec((D, D), lambda i: (0, 0)),
            pl.BlockSpec((1, D), lambda i: (0, 0)),
        ],
        out_specs=out_spec,
        out_shape=out_shape,
    )(z2, w, b.reshape(1, D))


def kernel(x, edge_index, W1, b1, W2, b2, W3, b3):
    ei = edge_index.astype(jnp.int32)
    pad = E_PAD - E
    src = jnp.concatenate([ei[0], jnp.zeros((pad,), jnp.int32)])
    dst = jnp.concatenate([ei[1], jnp.full((pad,), N, jnp.int32)])
    src = src.reshape(NCHUNK, CHUNK)
    dst = dst.reshape(NCHUNK, CHUNK)

    h2 = jnp.stack([x[:, :HALF], x[:, HALF:]])
    z2 = _agg(h2, src, dst)
    h2 = _mlp(z2, W1, b1, True, True)
    z2 = _agg(h2, src, dst)
    h2 = _mlp(z2, W2, b2, True, True)
    z2 = _agg(h2, src, dst)
    return _mlp(z2, W3, b3, False, False)


# speedup vs baseline: 2.0702x; 1.1226x over previous
import functools

import jax
import jax.numpy as jnp
from jax import lax
from jax.experimental import pallas as pl
from jax.experimental.pallas import tpu as pltpu
from jax.experimental.pallas import tpu_sc as plsc

N = 10000
D = 128
E = 320000
HALF = 64
NS = 16
CHUNK = 128
CPT = 160
NCHUNK = CPT * NS
E_PAD = NCHUNK * CHUNK
NBLK = 4
CPB = CPT // NBLK
ROWS_PT = N // NS
NBUF = 4


def _agg_body(h_hbm, src_hbm, dst_hbm, out_hbm,
              h_sh, agg_sh, sidx_v, didx_v,
              buf0, buf1, buf2, buf3, gsems, ssems):
    c = lax.axis_index("c")
    s = lax.axis_index("s")
    c0 = c * HALF
    r0 = s * ROWS_PT
    bufs = (buf0, buf1, buf2, buf3)

    pltpu.sync_copy(h_hbm.at[pl.ds(r0, ROWS_PT), pl.ds(c0, HALF)],
                    h_sh.at[pl.ds(r0, ROWS_PT)])
    pltpu.sync_copy(h_hbm.at[pl.ds(r0, ROWS_PT), pl.ds(c0, HALF)],
                    agg_sh.at[pl.ds(r0, ROWS_PT)])
    plsc.subcore_barrier()

    def gather(j, t):
        return pltpu.async_copy(h_sh.at[sidx_v.at[j]], bufs[t], gsems.at[t])

    def gather_wait(j, t):
        pltpu.make_async_copy(h_sh.at[sidx_v.at[j]], bufs[t],
                              gsems.at[t]).wait()

    def scatter(j, t):
        return pltpu.async_copy(bufs[t], agg_sh.at[didx_v.at[j]],
                                ssems.at[t], add=True)

    def scatter_wait(j, t):
        pltpu.make_async_copy(bufs[t], agg_sh.at[didx_v.at[j]],
                              ssems.at[t]).wait()

    def blk(bi, carry):
        ch0 = s * CPT + bi * CPB
        pltpu.sync_copy(src_hbm.at[pl.ds(ch0, CPB)], sidx_v)
        pltpu.sync_copy(dst_hbm.at[pl.ds(ch0, CPB)], didx_v)

        def body(q, carry2):
            j = NBUF * q
            for t in range(NBUF):
                gather(j + t, t)
            for t in range(NBUF):
                gather_wait(j + t, t)
                scatter(j + t, t)
            for t in range(NBUF):
                scatter_wait(j + t, t)
            return carry2

        lax.fori_loop(0, CPB // NBUF, body, 0)
        return carry

    lax.fori_loop(0, NBLK, blk, 0)
    plsc.subcore_barrier()

    pltpu.sync_copy(agg_sh.at[pl.ds(r0, ROWS_PT)],
                    out_hbm.at[pl.ds(r0, ROWS_PT), pl.ds(c0, HALF)])


_agg = pl.kernel(
    _agg_body,
    out_type=jax.ShapeDtypeStruct((N, D), jnp.float32),
    mesh=plsc.VectorSubcoreMesh(core_axis_name="c", subcore_axis_name="s"),
    scratch_types=[
        pltpu.VMEM_SHARED((N, HALF), jnp.float32),
        pltpu.VMEM_SHARED((N + 8, HALF), jnp.float32),
        pltpu.VMEM((CPB, CHUNK), jnp.int32),
        pltpu.VMEM((CPB, CHUNK), jnp.int32),
        pltpu.VMEM((CHUNK, HALF), jnp.float32),
        pltpu.VMEM((CHUNK, HALF), jnp.float32),
        pltpu.VMEM((CHUNK, HALF), jnp.float32),
        pltpu.VMEM((CHUNK, HALF), jnp.float32),
        pltpu.SemaphoreType.DMA((NBUF,)),
        pltpu.SemaphoreType.DMA((NBUF,)),
    ],
    compiler_params=pltpu.CompilerParams(use_tc_tiling_on_sc=False),
)


def _mlp_body(z_ref, w_ref, b_ref, o_ref, *, relu):
    acc = jnp.dot(z_ref[...], w_ref[...],
                  preferred_element_type=jnp.float32) + b_ref[...]
    o_ref[...] = jnp.maximum(acc, 0.0) if relu else acc


def _mlp(z, w, b, relu):
    blk = 1000
    return pl.pallas_call(
        functools.partial(_mlp_body, relu=relu),
        grid=(N // blk,),
        in_specs=[
            pl.BlockSpec((blk, D), lambda i: (i, 0)),
            pl.BlockSpec((D, D), lambda i: (0, 0)),
            pl.BlockSpec((1, D), lambda i: (0, 0)),
        ],
        out_specs=pl.BlockSpec((blk, D), lambda i: (i, 0)),
        out_shape=jax.ShapeDtypeStruct((N, D), jnp.float32),
    )(z, w, b.reshape(1, D))


def kernel(x, edge_index, W1, b1, W2, b2, W3, b3):
    ei = edge_index.astype(jnp.int32)
    pad = E_PAD - E
    src = jnp.concatenate([ei[0], jnp.zeros((pad,), jnp.int32)])
    dst = jnp.concatenate([ei[1], jnp.full((pad,), N, jnp.int32)])
    src = src.reshape(NCHUNK, CHUNK)
    dst = dst.reshape(NCHUNK, CHUNK)

    h = x
    z = _agg(h, src, dst)
    h = _mlp(z, W1, b1, True)
    z = _agg(h, src, dst)
    h = _mlp(z, W2, b2, True)
    z = _agg(h, src, dst)
    return _mlp(z, W3, b3, False)
